# baseline (speedup 1.0000x reference)
import functools

import jax
import jax.numpy as jnp
from jax import lax
from jax.experimental import pallas as pl
from jax.experimental.pallas import tpu as pltpu
from jax.experimental.pallas import tpu_sc as plsc

S, D, E = 2048, 768, 8
DFF = 4 * D
C = S
EC = E * C
BM = 128
NB = C // BM
NC, NS = 2, 16
NW = NC * NS
TPW = S // NW
GW = 128
NBLK = (2 * S) // BM + (E - 1)
NP = 128


def _gating_body(x_ref, wg_ref, pos1_ref, pos2_ref, g1_ref, g2_ref, cnt_ref,
                 blk_e_ref, blk_rb_ref):
    x = x_ref[...]
    wg = wg_ref[...]
    logits = lax.dot_general(x, wg, (((1,), (1,)), ((), ())),
                             preferred_element_type=jnp.float32)
    e_iota = lax.broadcasted_iota(jnp.int32, (S, E), 1)

    m1 = jnp.max(logits, axis=1, keepdims=True)
    i1 = jnp.min(jnp.where(logits == m1, e_iota, E), axis=1)
    masked = jnp.where(e_iota == i1[:, None], -jnp.inf, logits)
    m2 = jnp.max(masked, axis=1, keepdims=True)
    i2 = jnp.min(jnp.where(masked == m2, e_iota, E), axis=1)

    ex = jnp.exp(m2 - m1)
    g1 = 1.0 / (1.0 + ex)
    g2 = ex / (1.0 + ex)

    onehot = ((e_iota == i1[:, None]) | (e_iota == i2[:, None]))
    oh_bf = onehot.astype(jnp.bfloat16)
    r_iota = lax.broadcasted_iota(jnp.int32, (S, S), 0)
    c_iota = lax.broadcasted_iota(jnp.int32, (S, S), 1)
    tri = (c_iota < r_iota).astype(jnp.bfloat16)
    rank = lax.dot_general(tri, oh_bf, (((1,), (0,)), ((), ())),
                           preferred_element_type=jnp.float32)
    counts = jnp.sum(onehot.astype(jnp.float32), axis=0)

    rank1 = jnp.sum(jnp.where(e_iota == i1[:, None], rank, 0.0), axis=1)
    rank2 = jnp.sum(jnp.where(e_iota == i2[:, None], rank, 0.0), axis=1)
    pos1 = i1 * C + rank1.astype(jnp.int32)
    pos2 = i2 * C + rank2.astype(jnp.int32)

    cnt_i = counts.astype(jnp.int32)
    na = (cnt_i + (BM - 1)) // BM
    tri8 = (lax.broadcasted_iota(jnp.int32, (E, E), 0)
            <= lax.broadcasted_iota(jnp.int32, (E, E), 1)).astype(jnp.float32)
    incl = lax.dot_general(na.astype(jnp.float32)[None, :], tri8,
                           (((1,), (0,)), ((), ())),
                           preferred_element_type=jnp.float32)
    incl = incl.astype(jnp.int32)
    excl = incl - na[None, :]
    ii = lax.broadcasted_iota(jnp.int32, (NP, E), 0)
    e8 = lax.broadcasted_iota(jnp.int32, (NP, E), 1)
    e_of = jnp.sum((ii >= jnp.broadcast_to(incl, (NP, E))).astype(jnp.int32),
                   axis=1)
    blk_i = jnp.min(ii, axis=1)
    excl_sel = jnp.sum(jnp.where(e8 == e_of[:, None],
                                 jnp.broadcast_to(excl, (NP, E)), 0), axis=1)
    e_cl = jnp.minimum(e_of, E - 1)
    rb = e_cl * NB + (blk_i - excl_sel)
    total = jnp.max(incl)
    valid = blk_i < total
    rb_last = jnp.sum(jnp.where(blk_i == total - 1, rb, 0))
    blk_e = jnp.where(valid, e_cl, -1)
    blk_rb = jnp.where(valid, rb, rb_last)

    pos1_ref[...] = pos1[:, None]
    pos2_ref[...] = pos2[:, None]
    g1_ref[...] = jnp.broadcast_to(g1, (S, GW))
    g2_ref[...] = jnp.broadcast_to(g2, (S, GW))
    cnt_ref[...] = cnt_i[None, :]
    blk_e_ref[...] = blk_e[None, :]
    blk_rb_ref[...] = blk_rb[None, :]


def _gating(x2, wg):
    return pl.pallas_call(
        _gating_body,
        out_shape=(
            jax.ShapeDtypeStruct((S, 1), jnp.int32),
            jax.ShapeDtypeStruct((S, 1), jnp.int32),
            jax.ShapeDtypeStruct((S, GW), jnp.float32),
            jax.ShapeDtypeStruct((S, GW), jnp.float32),
            jax.ShapeDtypeStruct((1, E), jnp.int32),
            jax.ShapeDtypeStruct((1, NP), jnp.int32),
            jax.ShapeDtypeStruct((1, NP), jnp.int32),
        ),
    )(x2, wg)


@functools.cache
def _build_dispatch():
    mesh = plsc.VectorSubcoreMesh(core_axis_name="c", subcore_axis_name="s")

    @functools.partial(
        pl.kernel,
        mesh=mesh,
        out_type=(
            jax.ShapeDtypeStruct((EC, D), jnp.float32),
            jax.ShapeDtypeStruct((EC, GW), jnp.float32),
        ),
        scratch_types=[
            pltpu.VMEM((TPW, D), jnp.float32),
            pltpu.VMEM((TPW,), jnp.int32),
            pltpu.VMEM((TPW,), jnp.int32),
            pltpu.VMEM((TPW, GW), jnp.float32),
            pltpu.VMEM((TPW, GW), jnp.float32),
            pltpu.SemaphoreType.DMA,
            pltpu.SemaphoreType.DMA,
            pltpu.SemaphoreType.DMA,
            pltpu.SemaphoreType.DMA,
        ],
    )
    def dispatch(x_hbm, pos1_hbm, pos2_hbm, g1_hbm, g2_hbm, xs_hbm, gs_hbm,
                 rows_v, idx1_v, idx2_v, g1_v, g2_v, s1, s2, s3, s4):
        wid = lax.axis_index("s") * NC + lax.axis_index("c")
        base = wid * TPW
        pltpu.sync_copy(pos1_hbm.at[pl.ds(base, TPW)], idx1_v)
        pltpu.sync_copy(pos2_hbm.at[pl.ds(base, TPW)], idx2_v)
        pltpu.sync_copy(x_hbm.at[pl.ds(base, TPW)], rows_v)
        pltpu.sync_copy(g1_hbm.at[pl.ds(base, TPW)], g1_v)
        pltpu.sync_copy(g2_hbm.at[pl.ds(base, TPW)], g2_v)
        c1 = pltpu.async_copy(rows_v, xs_hbm.at[idx1_v], s1)
        c2 = pltpu.async_copy(rows_v, xs_hbm.at[idx2_v], s2)
        c3 = pltpu.async_copy(g1_v, gs_hbm.at[idx1_v], s3)
        c4 = pltpu.async_copy(g2_v, gs_hbm.at[idx2_v], s4)
        c1.wait()
        c2.wait()
        c3.wait()
        c4.wait()

    return dispatch


def _gmm_body(be_ref, rb_ref, x_ref, w1_ref, w2_ref, g_ref, o_ref):
    i = pl.program_id(0)

    @pl.when(be_ref[0, i] >= 0)
    def _():
        xb = x_ref[...]
        h = lax.dot_general(xb, w1_ref[0], (((1,), (1,)), ((), ())),
                            preferred_element_type=jnp.float32,
                            precision=lax.Precision.DEFAULT)
        h = jnp.maximum(h, 0.0)
        y = lax.dot_general(h, w2_ref[0], (((1,), (1,)), ((), ())),
                            preferred_element_type=jnp.float32,
                            precision=lax.Precision.DEFAULT)
        o_ref[...] = y * g_ref[...][:, :1]


def _gmm(blk_e, blk_rb, xs, w1, w2, gs):
    grid_spec = pltpu.PrefetchScalarGridSpec(
        num_scalar_prefetch=2,
        grid=(NBLK,),
        in_specs=[
            pl.BlockSpec((BM, D), lambda i, be, rb: (rb[0, i], 0)),
            pl.BlockSpec((1, DFF, D),
                         lambda i, be, rb: (jnp.maximum(be[0, i], 0), 0, 0)),
            pl.BlockSpec((1, D, DFF),
                         lambda i, be, rb: (jnp.maximum(be[0, i], 0), 0, 0)),
            pl.BlockSpec((BM, GW), lambda i, be, rb: (rb[0, i], 0)),
        ],
        out_specs=pl.BlockSpec((BM, D), lambda i, be, rb: (rb[0, i], 0)),
    )
    return pl.pallas_call(
        _gmm_body,
        grid_spec=grid_spec,
        out_shape=jax.ShapeDtypeStruct((EC, D), jnp.float32),
    )(blk_e, blk_rb, xs, w1, w2, gs)


@functools.cache
def _build_combine():
    mesh = plsc.VectorSubcoreMesh(core_axis_name="c", subcore_axis_name="s")

    @functools.partial(
        pl.kernel,
        mesh=mesh,
        out_type=jax.ShapeDtypeStruct((S, D), jnp.float32),
        scratch_types=[
            pltpu.VMEM((TPW,), jnp.int32),
            pltpu.VMEM((TPW,), jnp.int32),
            pltpu.VMEM((TPW, D), jnp.float32),
            pltpu.VMEM((TPW, D), jnp.float32),
            pltpu.SemaphoreType.DMA,
            pltpu.SemaphoreType.DMA,
        ],
    )
    def combine(ys_hbm, pos1_hbm, pos2_hbm, out_hbm,
                idx1_v, idx2_v, r1_v, r2_v, s1, s2):
        wid = lax.axis_index("s") * NC + lax.axis_index("c")
        base = wid * TPW
        pltpu.sync_copy(pos1_hbm.at[pl.ds(base, TPW)], idx1_v)
        pltpu.sync_copy(pos2_hbm.at[pl.ds(base, TPW)], idx2_v)
        c1 = pltpu.async_copy(ys_hbm.at[idx1_v], r1_v, s1)
        c2 = pltpu.async_copy(ys_hbm.at[idx2_v], r2_v, s2)
        c1.wait()
        c2.wait()

        @pl.loop(0, TPW)
        def _(r):
            @pl.loop(0, D, step=16)
            def _(c):
                sl = (pl.ds(r, 1), pl.ds(c, 16))
                r1_v[sl] = r1_v[sl] + r2_v[sl]

        pltpu.sync_copy(r1_v, out_hbm.at[pl.ds(base, TPW)])

    return combine


def kernel(x, Wg, W1, W2):
    x2 = x.reshape(S, D)
    pos1_2d, pos2_2d, g1r, g2r, cnt, blk_e, blk_rb = _gating(x2, Wg)
    pos1 = pos1_2d.reshape(S)
    pos2 = pos2_2d.reshape(S)
    xs, gs = _build_dispatch()(x2, pos1, pos2, g1r, g2r)
    ys = _gmm(blk_e, blk_rb, xs, W1, W2, gs)
    out = _build_combine()(ys, pos1, pos2)
    return out.reshape(1, S, D)


# speedup vs baseline: 1.4471x; 1.4471x over previous
import functools

import jax
import jax.numpy as jnp
from jax import lax
from jax.experimental import pallas as pl
from jax.experimental.pallas import tpu as pltpu
from jax.experimental.pallas import tpu_sc as plsc

S, D, E = 2048, 768, 8
DFF = 4 * D
C = S
EC = E * C
BM = 512
NB = C // BM
NC, NS = 2, 16
NW = NC * NS
TPW = S // NW
GW = 128
NBLK = (2 * S) // BM + (E - 1)
NP = 128


def _gating_body(x_ref, wg_ref, pos1_ref, pos2_ref, g1_ref, g2_ref, cnt_ref,
                 blk_e_ref, blk_rb_ref):
    x = x_ref[...]
    wg = wg_ref[...]
    logits = lax.dot_general(x, wg, (((1,), (1,)), ((), ())),
                             preferred_element_type=jnp.float32)
    e_iota = lax.broadcasted_iota(jnp.int32, (S, E), 1)

    m1 = jnp.max(logits, axis=1, keepdims=True)
    i1 = jnp.min(jnp.where(logits == m1, e_iota, E), axis=1)
    masked = jnp.where(e_iota == i1[:, None], -jnp.inf, logits)
    m2 = jnp.max(masked, axis=1, keepdims=True)
    i2 = jnp.min(jnp.where(masked == m2, e_iota, E), axis=1)

    ex = jnp.exp(m2 - m1)
    g1 = 1.0 / (1.0 + ex)
    g2 = ex / (1.0 + ex)

    onehot = ((e_iota == i1[:, None]) | (e_iota == i2[:, None]))
    oh_bf = onehot.astype(jnp.bfloat16)
    r_iota = lax.broadcasted_iota(jnp.int32, (S, S), 0)
    c_iota = lax.broadcasted_iota(jnp.int32, (S, S), 1)
    tri = (c_iota < r_iota).astype(jnp.bfloat16)
    rank = lax.dot_general(tri, oh_bf, (((1,), (0,)), ((), ())),
                           preferred_element_type=jnp.float32)
    counts = jnp.sum(onehot.astype(jnp.float32), axis=0)

    rank1 = jnp.sum(jnp.where(e_iota == i1[:, None], rank, 0.0), axis=1)
    rank2 = jnp.sum(jnp.where(e_iota == i2[:, None], rank, 0.0), axis=1)
    pos1 = i1 * C + rank1.astype(jnp.int32)
    pos2 = i2 * C + rank2.astype(jnp.int32)

    cnt_i = counts.astype(jnp.int32)
    na = (cnt_i + (BM - 1)) // BM
    tri8 = (lax.broadcasted_iota(jnp.int32, (E, E), 0)
            <= lax.broadcasted_iota(jnp.int32, (E, E), 1)).astype(jnp.float32)
    incl = lax.dot_general(na.astype(jnp.float32)[None, :], tri8,
                           (((1,), (0,)), ((), ())),
                           preferred_element_type=jnp.float32)
    incl = incl.astype(jnp.int32)
    excl = incl - na[None, :]
    ii = lax.broadcasted_iota(jnp.int32, (NP, E), 0)
    e8 = lax.broadcasted_iota(jnp.int32, (NP, E), 1)
    e_of = jnp.sum((ii >= jnp.broadcast_to(incl, (NP, E))).astype(jnp.int32),
                   axis=1)
    blk_i = jnp.min(ii, axis=1)
    excl_sel = jnp.sum(jnp.where(e8 == e_of[:, None],
                                 jnp.broadcast_to(excl, (NP, E)), 0), axis=1)
    e_cl = jnp.minimum(e_of, E - 1)
    rb = e_cl * NB + (blk_i - excl_sel)
    total = jnp.max(incl)
    valid = blk_i < total
    rb_last = jnp.sum(jnp.where(blk_i == total - 1, rb, 0))
    blk_e = jnp.where(valid, e_cl, -1)
    blk_rb = jnp.where(valid, rb, rb_last)

    pos1_ref[...] = pos1[:, None]
    pos2_ref[...] = pos2[:, None]
    g1_ref[...] = jnp.broadcast_to(g1, (S, GW))
    g2_ref[...] = jnp.broadcast_to(g2, (S, GW))
    cnt_ref[...] = cnt_i[None, :]
    blk_e_ref[...] = blk_e[None, :]
    blk_rb_ref[...] = blk_rb[None, :]


def _gating(x2, wg):
    return pl.pallas_call(
        _gating_body,
        out_shape=(
            jax.ShapeDtypeStruct((S, 1), jnp.int32),
            jax.ShapeDtypeStruct((S, 1), jnp.int32),
            jax.ShapeDtypeStruct((S, GW), jnp.float32),
            jax.ShapeDtypeStruct((S, GW), jnp.float32),
            jax.ShapeDtypeStruct((1, E), jnp.int32),
            jax.ShapeDtypeStruct((1, NP), jnp.int32),
            jax.ShapeDtypeStruct((1, NP), jnp.int32),
        ),
    )(x2, wg)


@functools.cache
def _build_dispatch():
    mesh = plsc.VectorSubcoreMesh(core_axis_name="c", subcore_axis_name="s")

    @functools.partial(
        pl.kernel,
        mesh=mesh,
        out_type=(
            jax.ShapeDtypeStruct((EC, D), jnp.float32),
            jax.ShapeDtypeStruct((EC, GW), jnp.float32),
        ),
        scratch_types=[
            pltpu.VMEM((TPW, D), jnp.float32),
            pltpu.VMEM((TPW,), jnp.int32),
            pltpu.VMEM((TPW,), jnp.int32),
            pltpu.VMEM((TPW, GW), jnp.float32),
            pltpu.VMEM((TPW, GW), jnp.float32),
            pltpu.SemaphoreType.DMA,
            pltpu.SemaphoreType.DMA,
            pltpu.SemaphoreType.DMA,
            pltpu.SemaphoreType.DMA,
        ],
    )
    def dispatch(x_hbm, pos1_hbm, pos2_hbm, g1_hbm, g2_hbm, xs_hbm, gs_hbm,
                 rows_v, idx1_v, idx2_v, g1_v, g2_v, s1, s2, s3, s4):
        wid = lax.axis_index("s") * NC + lax.axis_index("c")
        base = wid * TPW
        pltpu.sync_copy(pos1_hbm.at[pl.ds(base, TPW)], idx1_v)
        pltpu.sync_copy(pos2_hbm.at[pl.ds(base, TPW)], idx2_v)
        pltpu.sync_copy(x_hbm.at[pl.ds(base, TPW)], rows_v)
        pltpu.sync_copy(g1_hbm.at[pl.ds(base, TPW)], g1_v)
        pltpu.sync_copy(g2_hbm.at[pl.ds(base, TPW)], g2_v)
        c1 = pltpu.async_copy(rows_v, xs_hbm.at[idx1_v], s1)
        c2 = pltpu.async_copy(rows_v, xs_hbm.at[idx2_v], s2)
        c3 = pltpu.async_copy(g1_v, gs_hbm.at[idx1_v], s3)
        c4 = pltpu.async_copy(g2_v, gs_hbm.at[idx2_v], s4)
        c1.wait()
        c2.wait()
        c3.wait()
        c4.wait()

    return dispatch


def _gmm_body(be_ref, rb_ref, x_ref, w1_ref, w2_ref, g_ref, o_ref):
    i = pl.program_id(0)

    @pl.when(be_ref[0, i] >= 0)
    def _():
        xb = x_ref[...]
        h = lax.dot_general(xb, w1_ref[0], (((1,), (1,)), ((), ())),
                            preferred_element_type=jnp.float32,
                            precision=lax.Precision.DEFAULT)
        h = jnp.maximum(h, 0.0)
        y = lax.dot_general(h, w2_ref[0], (((1,), (1,)), ((), ())),
                            preferred_element_type=jnp.float32,
                            precision=lax.Precision.DEFAULT)
        o_ref[...] = y * g_ref[...][:, :1]


def _gmm(blk_e, blk_rb, xs, w1, w2, gs):
    grid_spec = pltpu.PrefetchScalarGridSpec(
        num_scalar_prefetch=2,
        grid=(NBLK,),
        in_specs=[
            pl.BlockSpec((BM, D), lambda i, be, rb: (rb[0, i], 0)),
            pl.BlockSpec((1, DFF, D),
                         lambda i, be, rb: (jnp.maximum(be[0, i], 0), 0, 0)),
            pl.BlockSpec((1, D, DFF),
                         lambda i, be, rb: (jnp.maximum(be[0, i], 0), 0, 0)),
            pl.BlockSpec((BM, GW), lambda i, be, rb: (rb[0, i], 0)),
        ],
        out_specs=pl.BlockSpec((BM, D), lambda i, be, rb: (rb[0, i], 0)),
    )
    return pl.pallas_call(
        _gmm_body,
        grid_spec=grid_spec,
        out_shape=jax.ShapeDtypeStruct((EC, D), jnp.float32),
    )(blk_e, blk_rb, xs, w1, w2, gs)


@functools.cache
def _build_combine():
    mesh = plsc.VectorSubcoreMesh(core_axis_name="c", subcore_axis_name="s")

    @functools.partial(
        pl.kernel,
        mesh=mesh,
        out_type=jax.ShapeDtypeStruct((S, D), jnp.float32),
        scratch_types=[
            pltpu.VMEM((TPW,), jnp.int32),
            pltpu.VMEM((TPW,), jnp.int32),
            pltpu.VMEM((TPW, D), jnp.float32),
            pltpu.VMEM((TPW, D), jnp.float32),
            pltpu.SemaphoreType.DMA,
            pltpu.SemaphoreType.DMA,
        ],
    )
    def combine(ys_hbm, pos1_hbm, pos2_hbm, out_hbm,
                idx1_v, idx2_v, r1_v, r2_v, s1, s2):
        wid = lax.axis_index("s") * NC + lax.axis_index("c")
        base = wid * TPW
        pltpu.sync_copy(pos1_hbm.at[pl.ds(base, TPW)], idx1_v)
        pltpu.sync_copy(pos2_hbm.at[pl.ds(base, TPW)], idx2_v)
        c1 = pltpu.async_copy(ys_hbm.at[idx1_v], r1_v, s1)
        c2 = pltpu.async_copy(ys_hbm.at[idx2_v], r2_v, s2)
        c1.wait()
        c2.wait()

        @pl.loop(0, TPW)
        def _(r):
            @pl.loop(0, D, step=16)
            def _(c):
                sl = (pl.ds(r, 1), pl.ds(c, 16))
                r1_v[sl] = r1_v[sl] + r2_v[sl]

        pltpu.sync_copy(r1_v, out_hbm.at[pl.ds(base, TPW)])

    return combine


def kernel(x, Wg, W1, W2):
    x2 = x.reshape(S, D)
    pos1_2d, pos2_2d, g1r, g2r, cnt, blk_e, blk_rb = _gating(x2, Wg)
    pos1 = pos1_2d.reshape(S)
    pos2 = pos2_2d.reshape(S)
    xs, gs = _build_dispatch()(x2, pos1, pos2, g1r, g2r)
    ys = _gmm(blk_e, blk_rb, xs, W1, W2, gs)
    out = _build_combine()(ys, pos1, pos2)
    return out.reshape(1, S, D)


# speedup vs baseline: 1.5138x; 1.0460x over previous
import functools

import jax
import jax.numpy as jnp
from jax import lax
from jax.experimental import pallas as pl
from jax.experimental.pallas import tpu as pltpu
from jax.experimental.pallas import tpu_sc as plsc

S, D, E = 2048, 768, 8
DFF = 4 * D
C = S
EC = E * C
BM = 512
NB = C // BM
NC, NS = 2, 16
NW = NC * NS
TPW = S // NW
GW = 128
NBLK = (2 * S) // BM + (E - 1)
NP = 128


def _gating_body(x_ref, wg_ref, pos1_ref, pos2_ref, g1_ref, g2_ref, cnt_ref,
                 blk_e_ref, blk_rb_ref, blk_new_ref, blk_iss_ref,
                 blk_nxe_ref, blk_slt_ref):
    x = x_ref[...]
    wg = wg_ref[...]
    logits = lax.dot_general(x, wg, (((1,), (1,)), ((), ())),
                             preferred_element_type=jnp.float32)
    e_iota = lax.broadcasted_iota(jnp.int32, (S, E), 1)

    m1 = jnp.max(logits, axis=1, keepdims=True)
    i1 = jnp.min(jnp.where(logits == m1, e_iota, E), axis=1)
    masked = jnp.where(e_iota == i1[:, None], -jnp.inf, logits)
    m2 = jnp.max(masked, axis=1, keepdims=True)
    i2 = jnp.min(jnp.where(masked == m2, e_iota, E), axis=1)

    ex = jnp.exp(m2 - m1)
    g1 = 1.0 / (1.0 + ex)
    g2 = ex / (1.0 + ex)

    onehot = ((e_iota == i1[:, None]) | (e_iota == i2[:, None]))
    oh_bf = onehot.astype(jnp.bfloat16)
    r_iota = lax.broadcasted_iota(jnp.int32, (S, S), 0)
    c_iota = lax.broadcasted_iota(jnp.int32, (S, S), 1)
    tri = (c_iota < r_iota).astype(jnp.bfloat16)
    rank = lax.dot_general(tri, oh_bf, (((1,), (0,)), ((), ())),
                           preferred_element_type=jnp.float32)
    counts = jnp.sum(onehot.astype(jnp.float32), axis=0)

    rank1 = jnp.sum(jnp.where(e_iota == i1[:, None], rank, 0.0), axis=1)
    rank2 = jnp.sum(jnp.where(e_iota == i2[:, None], rank, 0.0), axis=1)
    pos1 = i1 * C + rank1.astype(jnp.int32)
    pos2 = i2 * C + rank2.astype(jnp.int32)

    cnt_i = counts.astype(jnp.int32)
    na = (cnt_i + (BM - 1)) // BM
    tri8 = (lax.broadcasted_iota(jnp.int32, (E, E), 0)
            <= lax.broadcasted_iota(jnp.int32, (E, E), 1)).astype(jnp.float32)
    incl = lax.dot_general(na.astype(jnp.float32)[None, :], tri8,
                           (((1,), (0,)), ((), ())),
                           preferred_element_type=jnp.float32)
    incl = incl.astype(jnp.int32)
    excl = incl - na[None, :]
    ii = lax.broadcasted_iota(jnp.int32, (NP, E), 0)
    e8 = lax.broadcasted_iota(jnp.int32, (NP, E), 1)
    e_of = jnp.sum((ii >= jnp.broadcast_to(incl, (NP, E))).astype(jnp.int32),
                   axis=1)
    blk_i = jnp.min(ii, axis=1)
    excl_sel = jnp.sum(jnp.where(e8 == e_of[:, None],
                                 jnp.broadcast_to(excl, (NP, E)), 0), axis=1)
    e_cl = jnp.minimum(e_of, E - 1)
    rb = e_cl * NB + (blk_i - excl_sel)
    total = jnp.max(incl)
    valid = blk_i < total
    rb_last = jnp.sum(jnp.where(blk_i == total - 1, rb, 0))
    blk_e = jnp.where(valid, e_cl, -1)
    blk_rb = jnp.where(valid, rb, rb_last)

    be_prev = jnp.concatenate([jnp.full((1,), -2, jnp.int32), blk_e[:-1]])
    ch = (valid & ((blk_i == 0) | (blk_e != be_prev))).astype(jnp.int32)
    trip = (lax.broadcasted_iota(jnp.int32, (NP, NP), 0)
            <= lax.broadcasted_iota(jnp.int32, (NP, NP), 1)).astype(jnp.bfloat16)
    rid = lax.dot_general(ch.astype(jnp.bfloat16)[None, :], trip,
                          (((1,), (0,)), ((), ())),
                          preferred_element_type=jnp.float32)[0].astype(jnp.int32) - 1
    nrun = jnp.max(rid) + 1
    wslot = jnp.where(valid, rid % 2, 0)
    r_iota = lax.broadcasted_iota(jnp.int32, (NP, NP), 1)
    m_first = (ch[:, None] * (rid[:, None] == r_iota))
    run_e = jnp.sum(m_first * jnp.maximum(blk_e, 0)[:, None], axis=0)
    nexte = jnp.sum((r_iota == (rid[:, None] + 1)).astype(jnp.int32)
                    * run_e[None, :], axis=1)
    issue = ch * ((rid + 1) < nrun).astype(jnp.int32)

    pos1_ref[...] = pos1[:, None]
    pos2_ref[...] = pos2[:, None]
    g1_ref[...] = jnp.broadcast_to(g1, (S, GW))
    g2_ref[...] = jnp.broadcast_to(g2, (S, GW))
    cnt_ref[...] = cnt_i[None, :]
    blk_e_ref[...] = blk_e[None, :]
    blk_rb_ref[...] = blk_rb[None, :]
    blk_new_ref[...] = ch[None, :]
    blk_iss_ref[...] = issue[None, :]
    blk_nxe_ref[...] = nexte[None, :]
    blk_slt_ref[...] = wslot[None, :]


def _gating(x2, wg):
    return pl.pallas_call(
        _gating_body,
        out_shape=(
            jax.ShapeDtypeStruct((S, 1), jnp.int32),
            jax.ShapeDtypeStruct((S, 1), jnp.int32),
            jax.ShapeDtypeStruct((S, GW), jnp.float32),
            jax.ShapeDtypeStruct((S, GW), jnp.float32),
            jax.ShapeDtypeStruct((1, E), jnp.int32),
            jax.ShapeDtypeStruct((1, NP), jnp.int32),
            jax.ShapeDtypeStruct((1, NP), jnp.int32),
            jax.ShapeDtypeStruct((1, NP), jnp.int32),
            jax.ShapeDtypeStruct((1, NP), jnp.int32),
            jax.ShapeDtypeStruct((1, NP), jnp.int32),
            jax.ShapeDtypeStruct((1, NP), jnp.int32),
        ),
    )(x2, wg)


@functools.cache
def _build_dispatch():
    mesh = plsc.VectorSubcoreMesh(core_axis_name="c", subcore_axis_name="s")

    @functools.partial(
        pl.kernel,
        mesh=mesh,
        out_type=(
            jax.ShapeDtypeStruct((EC, D), jnp.float32),
            jax.ShapeDtypeStruct((EC, GW), jnp.float32),
        ),
        scratch_types=[
            pltpu.VMEM((TPW, D), jnp.float32),
            pltpu.VMEM((TPW,), jnp.int32),
            pltpu.VMEM((TPW,), jnp.int32),
            pltpu.VMEM((TPW, GW), jnp.float32),
            pltpu.VMEM((TPW, GW), jnp.float32),
            pltpu.SemaphoreType.DMA,
            pltpu.SemaphoreType.DMA,
            pltpu.SemaphoreType.DMA,
            pltpu.SemaphoreType.DMA,
        ],
    )
    def dispatch(x_hbm, pos1_hbm, pos2_hbm, g1_hbm, g2_hbm, xs_hbm, gs_hbm,
                 rows_v, idx1_v, idx2_v, g1_v, g2_v, s1, s2, s3, s4):
        wid = lax.axis_index("s") * NC + lax.axis_index("c")
        base = wid * TPW
        pltpu.sync_copy(pos1_hbm.at[pl.ds(base, TPW)], idx1_v)
        pltpu.sync_copy(pos2_hbm.at[pl.ds(base, TPW)], idx2_v)
        pltpu.sync_copy(x_hbm.at[pl.ds(base, TPW)], rows_v)
        pltpu.sync_copy(g1_hbm.at[pl.ds(base, TPW)], g1_v)
        pltpu.sync_copy(g2_hbm.at[pl.ds(base, TPW)], g2_v)
        c1 = pltpu.async_copy(rows_v, xs_hbm.at[idx1_v], s1)
        c2 = pltpu.async_copy(rows_v, xs_hbm.at[idx2_v], s2)
        c3 = pltpu.async_copy(g1_v, gs_hbm.at[idx1_v], s3)
        c4 = pltpu.async_copy(g2_v, gs_hbm.at[idx2_v], s4)
        c1.wait()
        c2.wait()
        c3.wait()
        c4.wait()

    return dispatch


def _gmm_body(be_ref, rb_ref, new_ref, iss_ref, nxe_ref, slt_ref,
              x_ref, w1_hbm, w2_hbm, g_ref, o_ref, w1s, w2s, sems):
    i = pl.program_id(0)
    cs = slt_ref[0, i]

    @pl.when(i == 0)
    def _():
        e0 = jnp.maximum(be_ref[0, 0], 0)
        pltpu.make_async_copy(w1_hbm.at[e0], w1s.at[0], sems.at[0]).start()
        pltpu.make_async_copy(w2_hbm.at[e0], w2s.at[0], sems.at[0]).start()

    @pl.when(iss_ref[0, i] == 1)
    def _():
        ne = nxe_ref[0, i]
        ns = 1 - cs
        pltpu.make_async_copy(w1_hbm.at[ne], w1s.at[ns], sems.at[ns]).start()
        pltpu.make_async_copy(w2_hbm.at[ne], w2s.at[ns], sems.at[ns]).start()

    @pl.when(new_ref[0, i] == 1)
    def _():
        pltpu.make_async_copy(w1_hbm.at[0], w1s.at[cs], sems.at[cs]).wait()
        pltpu.make_async_copy(w2_hbm.at[0], w2s.at[cs], sems.at[cs]).wait()

    @pl.when(be_ref[0, i] >= 0)
    def _():
        xb = x_ref[...]
        h = lax.dot_general(xb, w1s[cs], (((1,), (1,)), ((), ())),
                            preferred_element_type=jnp.float32,
                            precision=lax.Precision.DEFAULT)
        h = jnp.maximum(h, 0.0)
        y = lax.dot_general(h, w2s[cs], (((1,), (1,)), ((), ())),
                            preferred_element_type=jnp.float32,
                            precision=lax.Precision.DEFAULT)
        o_ref[...] = y * g_ref[...][:, :1]


def _gmm(blk_e, blk_rb, blk_new, blk_iss, blk_nxe, blk_slt, xs, w1, w2, gs):
    grid_spec = pltpu.PrefetchScalarGridSpec(
        num_scalar_prefetch=6,
        grid=(NBLK,),
        in_specs=[
            pl.BlockSpec((BM, D), lambda i, *s: (s[1][0, i], 0)),
            pl.BlockSpec(memory_space=pltpu.MemorySpace.HBM),
            pl.BlockSpec(memory_space=pltpu.MemorySpace.HBM),
            pl.BlockSpec((BM, GW), lambda i, *s: (s[1][0, i], 0)),
        ],
        out_specs=pl.BlockSpec((BM, D), lambda i, *s: (s[1][0, i], 0)),
        scratch_shapes=[
            pltpu.VMEM((2, DFF, D), jnp.float32),
            pltpu.VMEM((2, D, DFF), jnp.float32),
            pltpu.SemaphoreType.DMA((2,)),
        ],
    )
    return pl.pallas_call(
        _gmm_body,
        grid_spec=grid_spec,
        out_shape=jax.ShapeDtypeStruct((EC, D), jnp.float32),
    )(blk_e, blk_rb, blk_new, blk_iss, blk_nxe, blk_slt, xs, w1, w2, gs)


@functools.cache
def _build_combine():
    mesh = plsc.VectorSubcoreMesh(core_axis_name="c", subcore_axis_name="s")

    @functools.partial(
        pl.kernel,
        mesh=mesh,
        out_type=jax.ShapeDtypeStruct((S, D), jnp.float32),
        scratch_types=[
            pltpu.VMEM((TPW,), jnp.int32),
            pltpu.VMEM((TPW,), jnp.int32),
            pltpu.VMEM((TPW, D), jnp.float32),
            pltpu.VMEM((TPW, D), jnp.float32),
            pltpu.SemaphoreType.DMA,
            pltpu.SemaphoreType.DMA,
        ],
    )
    def combine(ys_hbm, pos1_hbm, pos2_hbm, out_hbm,
                idx1_v, idx2_v, r1_v, r2_v, s1, s2):
        wid = lax.axis_index("s") * NC + lax.axis_index("c")
        base = wid * TPW
        pltpu.sync_copy(pos1_hbm.at[pl.ds(base, TPW)], idx1_v)
        pltpu.sync_copy(pos2_hbm.at[pl.ds(base, TPW)], idx2_v)
        c1 = pltpu.async_copy(ys_hbm.at[idx1_v], r1_v, s1)
        c2 = pltpu.async_copy(ys_hbm.at[idx2_v], r2_v, s2)
        c1.wait()
        c2.wait()

        @pl.loop(0, TPW)
        def _(r):
            @pl.loop(0, D, step=16)
            def _(c):
                sl = (pl.ds(r, 1), pl.ds(c, 16))
                r1_v[sl] = r1_v[sl] + r2_v[sl]

        pltpu.sync_copy(r1_v, out_hbm.at[pl.ds(base, TPW)])

    return combine


def kernel(x, Wg, W1, W2):
    x2 = x.reshape(S, D)
    (pos1_2d, pos2_2d, g1r, g2r, cnt, blk_e, blk_rb,
     blk_new, blk_iss, blk_nxe, blk_slt) = _gating(x2, Wg)
    pos1 = pos1_2d.reshape(S)
    pos2 = pos2_2d.reshape(S)
    xs, gs = _build_dispatch()(x2, pos1, pos2, g1r, g2r)
    ys = _gmm(blk_e, blk_rb, blk_new, blk_iss, blk_nxe, blk_slt,
              xs, W1, W2, gs)
    out = _build_combine()(ys, pos1, pos2)
    return out.reshape(1, S, D)


# speedup vs baseline: 1.5419x; 1.0186x over previous
import functools

import jax
import jax.numpy as jnp
from jax import lax
from jax.experimental import pallas as pl
from jax.experimental.pallas import tpu as pltpu
from jax.experimental.pallas import tpu_sc as plsc

S, D, E = 2048, 768, 8
DFF = 4 * D
C = S
EC = E * C
BM = 256
NB = C // BM
NC, NS = 2, 16
NW = NC * NS
TPW = S // NW
GW = 128
NBLK = (2 * S) // BM + (E - 1)
NP = 128


def _gating_body(x_ref, wg_ref, pos1_ref, pos2_ref, g1_ref, g2_ref, cnt_ref,
                 blk_e_ref, blk_rb_ref, blk_new_ref, blk_iss_ref,
                 blk_nxe_ref, blk_slt_ref):
    x = x_ref[...]
    wg = wg_ref[...]
    logits = lax.dot_general(x, wg, (((1,), (1,)), ((), ())),
                             preferred_element_type=jnp.float32)
    e_iota = lax.broadcasted_iota(jnp.int32, (S, E), 1)

    m1 = jnp.max(logits, axis=1, keepdims=True)
    i1 = jnp.min(jnp.where(logits == m1, e_iota, E), axis=1)
    masked = jnp.where(e_iota == i1[:, None], -jnp.inf, logits)
    m2 = jnp.max(masked, axis=1, keepdims=True)
    i2 = jnp.min(jnp.where(masked == m2, e_iota, E), axis=1)

    ex = jnp.exp(m2 - m1)
    g1 = 1.0 / (1.0 + ex)
    g2 = ex / (1.0 + ex)

    onehot = ((e_iota == i1[:, None]) | (e_iota == i2[:, None]))
    oh_bf = onehot.astype(jnp.bfloat16)
    r_iota = lax.broadcasted_iota(jnp.int32, (S, S), 0)
    c_iota = lax.broadcasted_iota(jnp.int32, (S, S), 1)
    tri = (c_iota < r_iota).astype(jnp.bfloat16)
    rank = lax.dot_general(tri, oh_bf, (((1,), (0,)), ((), ())),
                           preferred_element_type=jnp.float32)
    counts = jnp.sum(onehot.astype(jnp.float32), axis=0)

    rank1 = jnp.sum(jnp.where(e_iota == i1[:, None], rank, 0.0), axis=1)
    rank2 = jnp.sum(jnp.where(e_iota == i2[:, None], rank, 0.0), axis=1)
    pos1 = i1 * C + rank1.astype(jnp.int32)
    pos2 = i2 * C + rank2.astype(jnp.int32)

    cnt_i = counts.astype(jnp.int32)
    na = (cnt_i + (BM - 1)) // BM
    tri8 = (lax.broadcasted_iota(jnp.int32, (E, E), 0)
            <= lax.broadcasted_iota(jnp.int32, (E, E), 1)).astype(jnp.float32)
    incl = lax.dot_general(na.astype(jnp.float32)[None, :], tri8,
                           (((1,), (0,)), ((), ())),
                           preferred_element_type=jnp.float32)
    incl = incl.astype(jnp.int32)
    excl = incl - na[None, :]
    ii = lax.broadcasted_iota(jnp.int32, (NP, E), 0)
    e8 = lax.broadcasted_iota(jnp.int32, (NP, E), 1)
    e_of = jnp.sum((ii >= jnp.broadcast_to(incl, (NP, E))).astype(jnp.int32),
                   axis=1)
    blk_i = jnp.min(ii, axis=1)
    excl_sel = jnp.sum(jnp.where(e8 == e_of[:, None],
                                 jnp.broadcast_to(excl, (NP, E)), 0), axis=1)
    e_cl = jnp.minimum(e_of, E - 1)
    rb = e_cl * NB + (blk_i - excl_sel)
    total = jnp.max(incl)
    valid = blk_i < total
    rb_last = jnp.sum(jnp.where(blk_i == total - 1, rb, 0))
    blk_e = jnp.where(valid, e_cl, -1)
    blk_rb = jnp.where(valid, rb, rb_last)

    be_prev = jnp.concatenate([jnp.full((1,), -2, jnp.int32), blk_e[:-1]])
    ch = (valid & ((blk_i == 0) | (blk_e != be_prev))).astype(jnp.int32)
    trip = (lax.broadcasted_iota(jnp.int32, (NP, NP), 0)
            <= lax.broadcasted_iota(jnp.int32, (NP, NP), 1)).astype(jnp.bfloat16)
    rid = lax.dot_general(ch.astype(jnp.bfloat16)[None, :], trip,
                          (((1,), (0,)), ((), ())),
                          preferred_element_type=jnp.float32)[0].astype(jnp.int32) - 1
    nrun = jnp.max(rid) + 1
    wslot = jnp.where(valid, rid % 2, 0)
    r_iota = lax.broadcasted_iota(jnp.int32, (NP, NP), 1)
    m_first = (ch[:, None] * (rid[:, None] == r_iota))
    run_e = jnp.sum(m_first * jnp.maximum(blk_e, 0)[:, None], axis=0)
    nexte = jnp.sum((r_iota == (rid[:, None] + 1)).astype(jnp.int32)
                    * run_e[None, :], axis=1)
    issue = ch * ((rid + 1) < nrun).astype(jnp.int32)

    pos1_ref[...] = pos1[:, None]
    pos2_ref[...] = pos2[:, None]
    g1_ref[...] = jnp.broadcast_to(g1, (S, GW))
    g2_ref[...] = jnp.broadcast_to(g2, (S, GW))
    cnt_ref[...] = cnt_i[None, :]
    blk_e_ref[...] = blk_e[None, :]
    blk_rb_ref[...] = blk_rb[None, :]
    blk_new_ref[...] = ch[None, :]
    blk_iss_ref[...] = issue[None, :]
    blk_nxe_ref[...] = nexte[None, :]
    blk_slt_ref[...] = wslot[None, :]


def _gating(x2, wg):
    return pl.pallas_call(
        _gating_body,
        out_shape=(
            jax.ShapeDtypeStruct((S, 1), jnp.int32),
            jax.ShapeDtypeStruct((S, 1), jnp.int32),
            jax.ShapeDtypeStruct((S, GW), jnp.float32),
            jax.ShapeDtypeStruct((S, GW), jnp.float32),
            jax.ShapeDtypeStruct((1, E), jnp.int32),
            jax.ShapeDtypeStruct((1, NP), jnp.int32),
            jax.ShapeDtypeStruct((1, NP), jnp.int32),
            jax.ShapeDtypeStruct((1, NP), jnp.int32),
            jax.ShapeDtypeStruct((1, NP), jnp.int32),
            jax.ShapeDtypeStruct((1, NP), jnp.int32),
            jax.ShapeDtypeStruct((1, NP), jnp.int32),
        ),
    )(x2, wg)


@functools.cache
def _build_dispatch():
    mesh = plsc.VectorSubcoreMesh(core_axis_name="c", subcore_axis_name="s")

    @functools.partial(
        pl.kernel,
        mesh=mesh,
        out_type=(
            jax.ShapeDtypeStruct((EC, D), jnp.float32),
            jax.ShapeDtypeStruct((EC, GW), jnp.float32),
        ),
        scratch_types=[
            pltpu.VMEM((TPW, D), jnp.float32),
            pltpu.VMEM((TPW,), jnp.int32),
            pltpu.VMEM((TPW,), jnp.int32),
            pltpu.VMEM((TPW, GW), jnp.float32),
            pltpu.VMEM((TPW, GW), jnp.float32),
            pltpu.SemaphoreType.DMA,
            pltpu.SemaphoreType.DMA,
            pltpu.SemaphoreType.DMA,
            pltpu.SemaphoreType.DMA,
        ],
    )
    def dispatch(x_hbm, pos1_hbm, pos2_hbm, g1_hbm, g2_hbm, xs_hbm, gs_hbm,
                 rows_v, idx1_v, idx2_v, g1_v, g2_v, s1, s2, s3, s4):
        wid = lax.axis_index("s") * NC + lax.axis_index("c")
        base = wid * TPW
        pltpu.sync_copy(pos1_hbm.at[pl.ds(base, TPW)], idx1_v)
        pltpu.sync_copy(pos2_hbm.at[pl.ds(base, TPW)], idx2_v)
        pltpu.sync_copy(x_hbm.at[pl.ds(base, TPW)], rows_v)
        pltpu.sync_copy(g1_hbm.at[pl.ds(base, TPW)], g1_v)
        pltpu.sync_copy(g2_hbm.at[pl.ds(base, TPW)], g2_v)
        c1 = pltpu.async_copy(rows_v, xs_hbm.at[idx1_v], s1)
        c2 = pltpu.async_copy(rows_v, xs_hbm.at[idx2_v], s2)
        c3 = pltpu.async_copy(g1_v, gs_hbm.at[idx1_v], s3)
        c4 = pltpu.async_copy(g2_v, gs_hbm.at[idx2_v], s4)
        c1.wait()
        c2.wait()
        c3.wait()
        c4.wait()

    return dispatch


def _gmm_body(be_ref, rb_ref, new_ref, iss_ref, nxe_ref, slt_ref,
              x_ref, w1_hbm, w2_hbm, g_ref, o_ref, w1s, w2s, sems):
    i = pl.program_id(0)
    cs = slt_ref[0, i]

    @pl.when(i == 0)
    def _():
        e0 = jnp.maximum(be_ref[0, 0], 0)
        pltpu.make_async_copy(w1_hbm.at[e0], w1s.at[0], sems.at[0]).start()
        pltpu.make_async_copy(w2_hbm.at[e0], w2s.at[0], sems.at[0]).start()

    @pl.when(iss_ref[0, i] == 1)
    def _():
        ne = nxe_ref[0, i]
        ns = 1 - cs
        pltpu.make_async_copy(w1_hbm.at[ne], w1s.at[ns], sems.at[ns]).start()
        pltpu.make_async_copy(w2_hbm.at[ne], w2s.at[ns], sems.at[ns]).start()

    @pl.when(new_ref[0, i] == 1)
    def _():
        pltpu.make_async_copy(w1_hbm.at[0], w1s.at[cs], sems.at[cs]).wait()
        pltpu.make_async_copy(w2_hbm.at[0], w2s.at[cs], sems.at[cs]).wait()

    @pl.when(be_ref[0, i] >= 0)
    def _():
        xb = x_ref[...]
        h = lax.dot_general(xb, w1s[cs], (((1,), (1,)), ((), ())),
                            preferred_element_type=jnp.float32,
                            precision=lax.Precision.DEFAULT)
        h = jnp.maximum(h, 0.0)
        y = lax.dot_general(h, w2s[cs], (((1,), (1,)), ((), ())),
                            preferred_element_type=jnp.float32,
                            precision=lax.Precision.DEFAULT)
        o_ref[...] = y * g_ref[...][:, :1]


def _gmm(blk_e, blk_rb, blk_new, blk_iss, blk_nxe, blk_slt, xs, w1, w2, gs):
    grid_spec = pltpu.PrefetchScalarGridSpec(
        num_scalar_prefetch=6,
        grid=(NBLK,),
        in_specs=[
            pl.BlockSpec((BM, D), lambda i, *s: (s[1][0, i], 0)),
            pl.BlockSpec(memory_space=pltpu.MemorySpace.HBM),
            pl.BlockSpec(memory_space=pltpu.MemorySpace.HBM),
            pl.BlockSpec((BM, GW), lambda i, *s: (s[1][0, i], 0)),
        ],
        out_specs=pl.BlockSpec((BM, D), lambda i, *s: (s[1][0, i], 0)),
        scratch_shapes=[
            pltpu.VMEM((2, DFF, D), jnp.float32),
            pltpu.VMEM((2, D, DFF), jnp.float32),
            pltpu.SemaphoreType.DMA((2,)),
        ],
    )
    return pl.pallas_call(
        _gmm_body,
        grid_spec=grid_spec,
        out_shape=jax.ShapeDtypeStruct((EC, D), jnp.float32),
    )(blk_e, blk_rb, blk_new, blk_iss, blk_nxe, blk_slt, xs, w1, w2, gs)


@functools.cache
def _build_combine():
    mesh = plsc.VectorSubcoreMesh(core_axis_name="c", subcore_axis_name="s")

    @functools.partial(
        pl.kernel,
        mesh=mesh,
        out_type=jax.ShapeDtypeStruct((S, D), jnp.float32),
        scratch_types=[
            pltpu.VMEM((TPW,), jnp.int32),
            pltpu.VMEM((TPW,), jnp.int32),
            pltpu.VMEM((TPW, D), jnp.float32),
            pltpu.VMEM((TPW, D), jnp.float32),
            pltpu.SemaphoreType.DMA,
            pltpu.SemaphoreType.DMA,
        ],
    )
    def combine(ys_hbm, pos1_hbm, pos2_hbm, out_hbm,
                idx1_v, idx2_v, r1_v, r2_v, s1, s2):
        wid = lax.axis_index("s") * NC + lax.axis_index("c")
        base = wid * TPW
        pltpu.sync_copy(pos1_hbm.at[pl.ds(base, TPW)], idx1_v)
        pltpu.sync_copy(pos2_hbm.at[pl.ds(base, TPW)], idx2_v)
        c1 = pltpu.async_copy(ys_hbm.at[idx1_v], r1_v, s1)
        c2 = pltpu.async_copy(ys_hbm.at[idx2_v], r2_v, s2)
        c1.wait()
        c2.wait()

        @pl.loop(0, TPW)
        def _(r):
            @pl.loop(0, D, step=16)
            def _(c):
                sl = (pl.ds(r, 1), pl.ds(c, 16))
                r1_v[sl] = r1_v[sl] + r2_v[sl]

        pltpu.sync_copy(r1_v, out_hbm.at[pl.ds(base, TPW)])

    return combine


def kernel(x, Wg, W1, W2):
    x2 = x.reshape(S, D)
    (pos1_2d, pos2_2d, g1r, g2r, cnt, blk_e, blk_rb,
     blk_new, blk_iss, blk_nxe, blk_slt) = _gating(x2, Wg)
    pos1 = pos1_2d.reshape(S)
    pos2 = pos2_2d.reshape(S)
    xs, gs = _build_dispatch()(x2, pos1, pos2, g1r, g2r)
    ys = _gmm(blk_e, blk_rb, blk_new, blk_iss, blk_nxe, blk_slt,
              xs, W1, W2, gs)
    out = _build_combine()(ys, pos1, pos2)
    return out.reshape(1, S, D)
